# baseline (speedup 1.0000x reference)
import functools

import jax
import jax.numpy as jnp
from jax import lax
from jax.experimental import pallas as pl
from jax.experimental.pallas import tpu as pltpu
from jax.experimental.pallas import tpu_sc as plsc

N_G = 10000
NPAD = 10240
BATCH = 128
N_EDGES = 320000
NW = 32
NCH = 80
CHUNK = 128
EPT = NCH * CHUNK
EPAD = NW * EPT
ROWS_PT = NPAD // 16

_MESH = plsc.VectorSubcoreMesh(core_axis_name="c", subcore_axis_name="s")


NSEC = 2
SCH = NCH // NSEC


@functools.partial(
    pl.kernel,
    out_type=jax.ShapeDtypeStruct((2, NPAD, BATCH), jnp.float32),
    mesh=_MESH,
    scratch_types=[
        pltpu.VMEM((SCH, CHUNK), jnp.int32),
        pltpu.VMEM((SCH, CHUNK), jnp.int32),
        pltpu.VMEM((2, CHUNK, BATCH), jnp.float32),
        pltpu.VMEM_SHARED((NPAD, BATCH), jnp.float32),
        pltpu.SemaphoreType.DMA,
        pltpu.SemaphoreType.DMA,
    ],
)
def _agg_sc(xs_hbm, src_hbm, dst_hbm, out_hbm, srcv, dstv, rows, acc,
            sem0, sem1):
    cid = lax.axis_index("c")
    sid = lax.axis_index("s")
    wid = cid * 16 + sid
    sems = (sem0, sem1)
    bounce = rows.at[0]

    def _zero(i, _):
        for j in range(BATCH // 16):
            bounce[i, pl.ds(j * 16, 16)] = jnp.zeros((16,), jnp.float32)
        return _

    lax.fori_loop(0, CHUNK, _zero, None)
    for k in range(ROWS_PT // CHUNK):
        pltpu.sync_copy(bounce, acc.at[pl.ds(sid * ROWS_PT + k * CHUNK, CHUNK)])
    plsc.subcore_barrier()

    for sec in range(NSEC):
        pltpu.sync_copy(src_hbm.at[wid, pl.ds(sec * SCH, SCH)], srcv)
        pltpu.sync_copy(dst_hbm.at[wid, pl.ds(sec * SCH, SCH)], dstv)

        for b in range(2):
            pltpu.make_async_copy(
                xs_hbm.at[srcv.at[b]], rows.at[b], sems[b]).start()

        def _group(g, _):
            for b in range(2):
                j = g * 2 + b
                pltpu.make_async_copy(
                    xs_hbm.at[srcv.at[j]], rows.at[b], sems[b]).wait()
                pltpu.sync_copy(rows.at[b], acc.at[dstv.at[j]], add=True)

                @pl.when(j + 2 < SCH)
                def _():
                    pltpu.make_async_copy(
                        xs_hbm.at[srcv.at[j + 2]], rows.at[b], sems[b]).start()
            return _

        lax.fori_loop(0, SCH // 2, _group, None)

    plsc.subcore_barrier()
    for k in range(ROWS_PT // CHUNK):
        off = sid * ROWS_PT + k * CHUNK
        pltpu.sync_copy(acc.at[pl.ds(off, CHUNK)], bounce)
        pltpu.sync_copy(bounce, out_hbm.at[cid, pl.ds(off, CHUNK)])


def _celu(v):
    return jnp.where(v > 0, v, jnp.exp(jnp.minimum(v, 0.0)) - 1.0)


def _dinv(h0, h1):
    return lax.rsqrt(h0 + h1 + 1.0)


GB = 512


def _scale_body(x_ref, h0_ref, h1_ref, xs_ref):
    xt = jnp.swapaxes(x_ref[...], 0, 1)
    xs_ref[...] = xt * _dinv(h0_ref[...], h1_ref[...])


def _scale_call(xp, h0, h1):
    grid = (NPAD // GB,)
    return pl.pallas_call(
        _scale_body,
        grid=grid,
        in_specs=[
            pl.BlockSpec((BATCH, GB), lambda i: (0, i)),
            pl.BlockSpec((GB, 1), lambda i: (i, 0)),
            pl.BlockSpec((GB, 1), lambda i: (i, 0)),
        ],
        out_specs=pl.BlockSpec((GB, BATCH), lambda i: (i, 0)),
        out_shape=jax.ShapeDtypeStruct((NPAD, BATCH), jnp.float32),
    )(xp, h0, h1)


def _mid_body(a0_ref, a1_ref, xs_ref, h0_ref, h1_ref, w1_ref, b1_ref, w2_ref,
              o_ref):
    i = pl.program_id(0)
    dinv = _dinv(h0_ref[...], h1_ref[...])
    u = dinv * (a0_ref[...] + a1_ref[...] + xs_ref[...])
    hmid = _celu(jnp.dot(u, w1_ref[...], preferred_element_type=jnp.float32)
                 + b1_ref[...])
    p = jnp.dot(hmid, w2_ref[...], preferred_element_type=jnp.float32)
    rows = i * GB + lax.broadcasted_iota(jnp.int32, (GB, 1), 0)
    o_ref[...] = jnp.where(rows < N_G, dinv * p, 0.0)


def _mid_call(a0, a1, xs1, h0, h1, W1, b1r, W2):
    grid = (NPAD // GB,)
    mid = W1.shape[1]
    return pl.pallas_call(
        _mid_body,
        grid=grid,
        in_specs=[
            pl.BlockSpec((GB, BATCH), lambda i: (i, 0)),
            pl.BlockSpec((GB, BATCH), lambda i: (i, 0)),
            pl.BlockSpec((GB, BATCH), lambda i: (i, 0)),
            pl.BlockSpec((GB, 1), lambda i: (i, 0)),
            pl.BlockSpec((GB, 1), lambda i: (i, 0)),
            pl.BlockSpec((BATCH, mid), lambda i: (0, 0)),
            pl.BlockSpec((1, mid), lambda i: (0, 0)),
            pl.BlockSpec((mid, BATCH), lambda i: (0, 0)),
        ],
        out_specs=pl.BlockSpec((GB, BATCH), lambda i: (i, 0)),
        out_shape=jax.ShapeDtypeStruct((NPAD, BATCH), jnp.float32),
    )(a0, a1, xs1, h0, h1, W1, b1r, W2)


FB = 2000


def _final1_body(a0_ref, a1_ref, xs_ref, h0_ref, h1_ref, b2_ref, w_ref,
                 db_ref, z_ref):
    i = pl.program_id(0)
    dinv = _dinv(h0_ref[...], h1_ref[...])
    hg = _celu(dinv * (a0_ref[...] + a1_ref[...] + xs_ref[...]) + b2_ref[...])
    zpart = lax.dot_general(hg, w_ref[...], (((0,), (0,)), ((), ())),
                            preferred_element_type=jnp.float32)

    @pl.when(i == 0)
    def _():
        z_ref[...] = jnp.zeros_like(z_ref)

    z_ref[...] += zpart

    @pl.when(i == (N_G // FB) - 1)
    def _():
        z_ref[...] += db_ref[...]


def _final1_call(a0, a1, xs2, h0, h1, b2r, dr_W, dr_br):
    grid = (N_G // FB,)
    td = dr_W.shape[1]
    return pl.pallas_call(
        _final1_body,
        grid=grid,
        in_specs=[
            pl.BlockSpec((FB, BATCH), lambda i: (i, 0)),
            pl.BlockSpec((FB, BATCH), lambda i: (i, 0)),
            pl.BlockSpec((FB, BATCH), lambda i: (i, 0)),
            pl.BlockSpec((FB, 1), lambda i: (i, 0)),
            pl.BlockSpec((FB, 1), lambda i: (i, 0)),
            pl.BlockSpec((1, BATCH), lambda i: (0, 0)),
            pl.BlockSpec((FB, td), lambda i: (i, 0)),
            pl.BlockSpec((1, td), lambda i: (0, 0)),
        ],
        out_specs=pl.BlockSpec((BATCH, td), lambda i: (0, 0)),
        out_shape=jax.ShapeDtypeStruct((BATCH, td), jnp.float32),
    )(a0, a1, xs2, h0, h1, b2r, dr_W, dr_br)


def _final2_body(z_ref, w1_ref, b1_ref, w2_ref, b2_ref, y_ref):
    t = _celu(jnp.dot(z_ref[...], w1_ref[...],
                      preferred_element_type=jnp.float32) + b1_ref[...])
    y_ref[...] = _celu(jnp.dot(t, w2_ref[...],
                               preferred_element_type=jnp.float32) + b2_ref[...])


FB2 = 2048


def _final2_call(z, r1_W, r1_br, r2_W, r2_br):
    grid = (NPAD // FB2,)
    td = z.shape[1]
    rm = r1_W.shape[1]
    r2_Wp = jnp.pad(r2_W, ((0, 0), (0, NPAD - N_G)))
    r2_brp = jnp.pad(r2_br, ((0, 0), (0, NPAD - N_G)))
    yp = pl.pallas_call(
        _final2_body,
        grid=grid,
        in_specs=[
            pl.BlockSpec((BATCH, td), lambda i: (0, 0)),
            pl.BlockSpec((td, rm), lambda i: (0, 0)),
            pl.BlockSpec((1, rm), lambda i: (0, 0)),
            pl.BlockSpec((rm, FB2), lambda i: (0, i)),
            pl.BlockSpec((1, FB2), lambda i: (0, i)),
        ],
        out_specs=pl.BlockSpec((BATCH, FB2), lambda i: (0, i)),
        out_shape=jax.ShapeDtypeStruct((BATCH, NPAD), jnp.float32),
    )(z, r1_W, r1_br, r2_Wp, r2_brp)
    return yp[:, :N_G]


def kernel(x, edge_index, W1, b1, W2, b2, dr_W, dr_b, r1_W, r1_b, r2_W, r2_b):
    xp = jnp.pad(x, ((0, 0), (0, NPAD - N_G)))
    pad_n = EPAD - N_EDGES
    pad_idx = jnp.full((pad_n,), N_G, jnp.int32)
    src_p = jnp.concatenate([edge_index[0], pad_idx]).reshape(NW, NCH, CHUNK)
    dst_p = jnp.concatenate([edge_index[1], pad_idx]).reshape(NW, NCH, CHUNK)

    ones_tab = jnp.ones((NPAD, BATCH), jnp.float32)
    hist = _agg_sc(ones_tab, src_p, dst_p)
    h0 = hist[0, :, 0:1]
    h1 = hist[1, :, 0:1]

    xs1 = _scale_call(xp, h0, h1)
    acc1 = _agg_sc(xs1, src_p, dst_p)
    xs2 = _mid_call(acc1[0], acc1[1], xs1, h0, h1, W1,
                    b1.reshape(1, -1), W2)
    acc2 = _agg_sc(xs2, src_p, dst_p)

    z = _final1_call(acc2[0, :N_G], acc2[1, :N_G], xs2[:N_G],
                     h0[:N_G], h1[:N_G], b2.reshape(1, -1),
                     dr_W, dr_b.reshape(1, -1))
    y = _final2_call(z, r1_W, r1_b.reshape(1, -1), r2_W, r2_b.reshape(1, -1))
    return (z, y)


# speedup vs baseline: 1.4892x; 1.4892x over previous
import functools

import jax
import jax.numpy as jnp
from jax import lax
from jax.experimental import pallas as pl
from jax.experimental.pallas import tpu as pltpu
from jax.experimental.pallas import tpu_sc as plsc

N_G = 10000
NPAD = 10240
BATCH = 128
N_EDGES = 320000
NW = 32
NCH = 80
CHUNK = 128
EPT = NCH * CHUNK
EPAD = NW * EPT
ROWS_PT = NPAD // 16

_MESH = plsc.VectorSubcoreMesh(core_axis_name="c", subcore_axis_name="s")


@functools.partial(
    pl.kernel,
    out_type=jax.ShapeDtypeStruct((2, NPAD, BATCH), jnp.float32),
    mesh=_MESH,
    scratch_types=[
        pltpu.VMEM((NCH // 2, CHUNK), jnp.int32),
        pltpu.VMEM((CHUNK, BATCH), jnp.float32),
        pltpu.VMEM_SHARED((NPAD, BATCH), jnp.float32),
    ],
)
def _hist_sc(dst_hbm, out_hbm, dstv, blk, acc):
    cid = lax.axis_index("c")
    sid = lax.axis_index("s")
    wid = cid * 16 + sid

    def _zero(i, _):
        for j in range(BATCH // 16):
            blk[i, pl.ds(j * 16, 16)] = jnp.zeros((16,), jnp.float32)
        return _

    lax.fori_loop(0, CHUNK, _zero, None)
    for k in range(ROWS_PT // CHUNK):
        pltpu.sync_copy(blk, acc.at[pl.ds(sid * ROWS_PT + k * CHUNK, CHUNK)])

    def _fill(i, _):
        for j in range(BATCH // 16):
            blk[i, pl.ds(j * 16, 16)] = jnp.full((16,), 1.0, jnp.float32)
        return _

    lax.fori_loop(0, CHUNK, _fill, None)
    plsc.subcore_barrier()

    for sec in range(2):
        pltpu.sync_copy(dst_hbm.at[wid, pl.ds(sec * (NCH // 2), NCH // 2)],
                        dstv)

        def _scat(j, _):
            pltpu.sync_copy(blk, acc.at[dstv.at[j]], add=True)
            return _

        lax.fori_loop(0, NCH // 2, _scat, None)

    plsc.subcore_barrier()
    for k in range(ROWS_PT // CHUNK):
        off = sid * ROWS_PT + k * CHUNK
        pltpu.sync_copy(acc.at[pl.ds(off, CHUNK)], blk)
        pltpu.sync_copy(blk, out_hbm.at[cid, pl.ds(off, CHUNK)])


NSEC = 2
SCH = NCH // NSEC


@functools.partial(
    pl.kernel,
    out_type=jax.ShapeDtypeStruct((2, NPAD, BATCH), jnp.float32),
    mesh=_MESH,
    scratch_types=[
        pltpu.VMEM((SCH, CHUNK), jnp.int32),
        pltpu.VMEM((SCH, CHUNK), jnp.int32),
        pltpu.VMEM((2, CHUNK, BATCH), jnp.float32),
        pltpu.VMEM_SHARED((NPAD, BATCH), jnp.float32),
        pltpu.SemaphoreType.DMA,
        pltpu.SemaphoreType.DMA,
    ],
)
def _agg_sc(xs_hbm, src_hbm, dst_hbm, out_hbm, srcv, dstv, rows, acc,
            sem0, sem1):
    cid = lax.axis_index("c")
    sid = lax.axis_index("s")
    wid = cid * 16 + sid
    sems = (sem0, sem1)
    bounce = rows.at[0]

    def _zero(i, _):
        for j in range(BATCH // 16):
            bounce[i, pl.ds(j * 16, 16)] = jnp.zeros((16,), jnp.float32)
        return _

    lax.fori_loop(0, CHUNK, _zero, None)
    for k in range(ROWS_PT // CHUNK):
        pltpu.sync_copy(bounce, acc.at[pl.ds(sid * ROWS_PT + k * CHUNK, CHUNK)])
    plsc.subcore_barrier()

    for sec in range(NSEC):
        pltpu.sync_copy(src_hbm.at[wid, pl.ds(sec * SCH, SCH)], srcv)
        pltpu.sync_copy(dst_hbm.at[wid, pl.ds(sec * SCH, SCH)], dstv)

        for b in range(2):
            pltpu.make_async_copy(
                xs_hbm.at[srcv.at[b]], rows.at[b], sems[b]).start()

        def _group(g, _):
            for b in range(2):
                j = g * 2 + b
                pltpu.make_async_copy(
                    xs_hbm.at[srcv.at[j]], rows.at[b], sems[b]).wait()
                pltpu.sync_copy(rows.at[b], acc.at[dstv.at[j]], add=True)

                @pl.when(j + 2 < SCH)
                def _():
                    pltpu.make_async_copy(
                        xs_hbm.at[srcv.at[j + 2]], rows.at[b], sems[b]).start()
            return _

        lax.fori_loop(0, SCH // 2, _group, None)

    plsc.subcore_barrier()
    for k in range(ROWS_PT // CHUNK):
        off = sid * ROWS_PT + k * CHUNK
        pltpu.sync_copy(acc.at[pl.ds(off, CHUNK)], bounce)
        pltpu.sync_copy(bounce, out_hbm.at[cid, pl.ds(off, CHUNK)])


def _celu(v):
    return jnp.where(v > 0, v, jnp.exp(jnp.minimum(v, 0.0)) - 1.0)


def _dinv(h0, h1):
    return lax.rsqrt(h0 + h1 + 1.0)


GB = 512


def _scale_body(x_ref, h0_ref, h1_ref, xs_ref):
    xt = jnp.swapaxes(x_ref[...], 0, 1)
    xs_ref[...] = xt * _dinv(h0_ref[...], h1_ref[...])


def _scale_call(xp, h0, h1):
    grid = (NPAD // GB,)
    return pl.pallas_call(
        _scale_body,
        grid=grid,
        in_specs=[
            pl.BlockSpec((BATCH, GB), lambda i: (0, i)),
            pl.BlockSpec((GB, 1), lambda i: (i, 0)),
            pl.BlockSpec((GB, 1), lambda i: (i, 0)),
        ],
        out_specs=pl.BlockSpec((GB, BATCH), lambda i: (i, 0)),
        out_shape=jax.ShapeDtypeStruct((NPAD, BATCH), jnp.float32),
    )(xp, h0, h1)


def _mid_body(a0_ref, a1_ref, xs_ref, h0_ref, h1_ref, w1_ref, b1_ref, w2_ref,
              o_ref):
    i = pl.program_id(0)
    dinv = _dinv(h0_ref[...], h1_ref[...])
    u = dinv * (a0_ref[...] + a1_ref[...] + xs_ref[...])
    hmid = _celu(jnp.dot(u, w1_ref[...], preferred_element_type=jnp.float32)
                 + b1_ref[...])
    p = jnp.dot(hmid, w2_ref[...], preferred_element_type=jnp.float32)
    rows = i * GB + lax.broadcasted_iota(jnp.int32, (GB, 1), 0)
    o_ref[...] = jnp.where(rows < N_G, dinv * p, 0.0)


def _mid_call(a0, a1, xs1, h0, h1, W1, b1r, W2):
    grid = (NPAD // GB,)
    mid = W1.shape[1]
    return pl.pallas_call(
        _mid_body,
        grid=grid,
        in_specs=[
            pl.BlockSpec((GB, BATCH), lambda i: (i, 0)),
            pl.BlockSpec((GB, BATCH), lambda i: (i, 0)),
            pl.BlockSpec((GB, BATCH), lambda i: (i, 0)),
            pl.BlockSpec((GB, 1), lambda i: (i, 0)),
            pl.BlockSpec((GB, 1), lambda i: (i, 0)),
            pl.BlockSpec((BATCH, mid), lambda i: (0, 0)),
            pl.BlockSpec((1, mid), lambda i: (0, 0)),
            pl.BlockSpec((mid, BATCH), lambda i: (0, 0)),
        ],
        out_specs=pl.BlockSpec((GB, BATCH), lambda i: (i, 0)),
        out_shape=jax.ShapeDtypeStruct((NPAD, BATCH), jnp.float32),
    )(a0, a1, xs1, h0, h1, W1, b1r, W2)


FB = 2000


def _final1_body(a0_ref, a1_ref, xs_ref, h0_ref, h1_ref, b2_ref, w_ref,
                 db_ref, z_ref):
    i = pl.program_id(0)
    dinv = _dinv(h0_ref[...], h1_ref[...])
    hg = _celu(dinv * (a0_ref[...] + a1_ref[...] + xs_ref[...]) + b2_ref[...])
    zpart = lax.dot_general(hg, w_ref[...], (((0,), (0,)), ((), ())),
                            preferred_element_type=jnp.float32)

    @pl.when(i == 0)
    def _():
        z_ref[...] = jnp.zeros_like(z_ref)

    z_ref[...] += zpart

    @pl.when(i == (N_G // FB) - 1)
    def _():
        z_ref[...] += db_ref[...]


def _final1_call(a0, a1, xs2, h0, h1, b2r, dr_W, dr_br):
    grid = (N_G // FB,)
    td = dr_W.shape[1]
    return pl.pallas_call(
        _final1_body,
        grid=grid,
        in_specs=[
            pl.BlockSpec((FB, BATCH), lambda i: (i, 0)),
            pl.BlockSpec((FB, BATCH), lambda i: (i, 0)),
            pl.BlockSpec((FB, BATCH), lambda i: (i, 0)),
            pl.BlockSpec((FB, 1), lambda i: (i, 0)),
            pl.BlockSpec((FB, 1), lambda i: (i, 0)),
            pl.BlockSpec((1, BATCH), lambda i: (0, 0)),
            pl.BlockSpec((FB, td), lambda i: (i, 0)),
            pl.BlockSpec((1, td), lambda i: (0, 0)),
        ],
        out_specs=pl.BlockSpec((BATCH, td), lambda i: (0, 0)),
        out_shape=jax.ShapeDtypeStruct((BATCH, td), jnp.float32),
    )(a0, a1, xs2, h0, h1, b2r, dr_W, dr_br)


def _final2_body(z_ref, w1_ref, b1_ref, w2_ref, b2_ref, y_ref):
    t = _celu(jnp.dot(z_ref[...], w1_ref[...],
                      preferred_element_type=jnp.float32) + b1_ref[...])
    y_ref[...] = _celu(jnp.dot(t, w2_ref[...],
                               preferred_element_type=jnp.float32) + b2_ref[...])


FB2 = 2048


def _final2_call(z, r1_W, r1_br, r2_W, r2_br):
    grid = (NPAD // FB2,)
    td = z.shape[1]
    rm = r1_W.shape[1]
    r2_Wp = jnp.pad(r2_W, ((0, 0), (0, NPAD - N_G)))
    r2_brp = jnp.pad(r2_br, ((0, 0), (0, NPAD - N_G)))
    yp = pl.pallas_call(
        _final2_body,
        grid=grid,
        in_specs=[
            pl.BlockSpec((BATCH, td), lambda i: (0, 0)),
            pl.BlockSpec((td, rm), lambda i: (0, 0)),
            pl.BlockSpec((1, rm), lambda i: (0, 0)),
            pl.BlockSpec((rm, FB2), lambda i: (0, i)),
            pl.BlockSpec((1, FB2), lambda i: (0, i)),
        ],
        out_specs=pl.BlockSpec((BATCH, FB2), lambda i: (0, i)),
        out_shape=jax.ShapeDtypeStruct((BATCH, NPAD), jnp.float32),
    )(z, r1_W, r1_br, r2_Wp, r2_brp)
    return yp[:, :N_G]


def kernel(x, edge_index, W1, b1, W2, b2, dr_W, dr_b, r1_W, r1_b, r2_W, r2_b):
    xp = jnp.pad(x, ((0, 0), (0, NPAD - N_G)))
    pad_n = EPAD - N_EDGES
    pad_idx = jnp.full((pad_n,), N_G, jnp.int32)
    src_p = jnp.concatenate([edge_index[0], pad_idx]).reshape(NW, NCH, CHUNK)
    dst_p = jnp.concatenate([edge_index[1], pad_idx]).reshape(NW, NCH, CHUNK)

    hist = _hist_sc(dst_p)
    h0 = hist[0, :, 0:1]
    h1 = hist[1, :, 0:1]

    xs1 = _scale_call(xp, h0, h1)
    acc1 = _agg_sc(xs1, src_p, dst_p)
    xs2 = _mid_call(acc1[0], acc1[1], xs1, h0, h1, W1,
                    b1.reshape(1, -1), W2)
    acc2 = _agg_sc(xs2, src_p, dst_p)

    z = _final1_call(acc2[0, :N_G], acc2[1, :N_G], xs2[:N_G],
                     h0[:N_G], h1[:N_G], b2.reshape(1, -1),
                     dr_W, dr_b.reshape(1, -1))
    y = _final2_call(z, r1_W, r1_b.reshape(1, -1), r2_W, r2_b.reshape(1, -1))
    return (z, y)


# speedup vs baseline: 1.5996x; 1.0741x over previous
import functools

import jax
import jax.numpy as jnp
from jax import lax
from jax.experimental import pallas as pl
from jax.experimental.pallas import tpu as pltpu
from jax.experimental.pallas import tpu_sc as plsc

N_G = 10000
NPAD = 10240
BATCH = 128
N_EDGES = 320000
NW = 32
NCH = 80
CHUNK = 128
EPT = NCH * CHUNK
EPAD = NW * EPT
ROWS_PT = NPAD // 16

_MESH = plsc.VectorSubcoreMesh(core_axis_name="c", subcore_axis_name="s")


@functools.partial(
    pl.kernel,
    out_type=jax.ShapeDtypeStruct((2, NPAD, BATCH), jnp.float32),
    mesh=_MESH,
    scratch_types=[
        pltpu.VMEM((NCH // 2, CHUNK), jnp.int32),
        pltpu.VMEM((CHUNK, BATCH), jnp.float32),
        pltpu.VMEM_SHARED((NPAD, BATCH), jnp.float32),
    ],
)
def _hist_sc(dst_hbm, out_hbm, dstv, blk, acc):
    cid = lax.axis_index("c")
    sid = lax.axis_index("s")
    wid = cid * 16 + sid

    def _zero(i, _):
        for j in range(BATCH // 16):
            blk[i, pl.ds(j * 16, 16)] = jnp.zeros((16,), jnp.float32)
        return _

    lax.fori_loop(0, CHUNK, _zero, None)
    for k in range(ROWS_PT // CHUNK):
        pltpu.sync_copy(blk, acc.at[pl.ds(sid * ROWS_PT + k * CHUNK, CHUNK)])

    def _fill(i, _):
        for j in range(BATCH // 16):
            blk[i, pl.ds(j * 16, 16)] = jnp.full((16,), 1.0, jnp.float32)
        return _

    lax.fori_loop(0, CHUNK, _fill, None)
    plsc.subcore_barrier()

    for sec in range(2):
        pltpu.sync_copy(dst_hbm.at[wid, pl.ds(sec * (NCH // 2), NCH // 2)],
                        dstv)

        def _scat(j, _):
            pltpu.sync_copy(blk, acc.at[dstv.at[j]], add=True)
            return _

        lax.fori_loop(0, NCH // 2, _scat, None)

    plsc.subcore_barrier()
    for k in range(ROWS_PT // CHUNK):
        off = sid * ROWS_PT + k * CHUNK
        pltpu.sync_copy(acc.at[pl.ds(off, CHUNK)], blk)
        pltpu.sync_copy(blk, out_hbm.at[cid, pl.ds(off, CHUNK)])


NSEC = 2
SCH = NCH // NSEC


@functools.partial(
    pl.kernel,
    out_type=jax.ShapeDtypeStruct((2, NPAD, BATCH), jnp.float32),
    mesh=_MESH,
    scratch_types=[
        pltpu.VMEM((SCH, CHUNK), jnp.int32),
        pltpu.VMEM((SCH, CHUNK), jnp.int32),
        pltpu.VMEM((2, CHUNK, BATCH), jnp.float32),
        pltpu.VMEM_SHARED((NPAD, BATCH), jnp.float32),
        pltpu.SemaphoreType.DMA,
        pltpu.SemaphoreType.DMA,
    ],
)
def _agg_sc(xs_hbm, src_hbm, dst_hbm, out_hbm, srcv, dstv, rows, acc,
            sem0, sem1):
    cid = lax.axis_index("c")
    sid = lax.axis_index("s")
    wid = cid * 16 + sid
    sems = (sem0, sem1)
    bounce = rows.at[0]

    def _zero(i, _):
        for j in range(BATCH // 16):
            bounce[i, pl.ds(j * 16, 16)] = jnp.zeros((16,), jnp.float32)
        return _

    lax.fori_loop(0, CHUNK, _zero, None)
    for k in range(ROWS_PT // CHUNK):
        pltpu.sync_copy(bounce, acc.at[pl.ds(sid * ROWS_PT + k * CHUNK, CHUNK)])
    plsc.subcore_barrier()

    for sec in range(NSEC):
        pltpu.sync_copy(src_hbm.at[wid, pl.ds(sec * SCH, SCH)], srcv)
        pltpu.sync_copy(dst_hbm.at[wid, pl.ds(sec * SCH, SCH)], dstv)

        for b in range(2):
            pltpu.make_async_copy(
                xs_hbm.at[srcv.at[b]], rows.at[b], sems[b]).start()

        def _group(g, _):
            for b in range(2):
                j = g * 2 + b
                pltpu.make_async_copy(
                    xs_hbm.at[srcv.at[j]], rows.at[b], sems[b]).wait()
                pltpu.sync_copy(rows.at[b], acc.at[dstv.at[j]], add=True)

                @pl.when(j + 2 < SCH)
                def _():
                    pltpu.make_async_copy(
                        xs_hbm.at[srcv.at[j + 2]], rows.at[b], sems[b]).start()
            return _

        lax.fori_loop(0, SCH // 2, _group, None)

    plsc.subcore_barrier()
    for k in range(ROWS_PT // CHUNK):
        off = sid * ROWS_PT + k * CHUNK
        pltpu.sync_copy(acc.at[pl.ds(off, CHUNK)], bounce)
        pltpu.sync_copy(bounce, out_hbm.at[cid, pl.ds(off, CHUNK)])


def _celu(v):
    return jnp.where(v > 0, v, jnp.exp(jnp.minimum(v, 0.0)) - 1.0)


def _dinv(h0, h1):
    return lax.rsqrt(h0 + h1 + 1.0)


GB = 512


def _scale_body(x_ref, h0_ref, h1_ref, xs_ref):
    xt = jnp.swapaxes(x_ref[...], 0, 1)
    xs_ref[...] = xt * _dinv(h0_ref[0, :, 0:1], h1_ref[0, :, 0:1])


def _scale_call(xp, hist):
    grid = (NPAD // GB,)
    return pl.pallas_call(
        _scale_body,
        grid=grid,
        in_specs=[
            pl.BlockSpec((BATCH, GB), lambda i: (0, i)),
            pl.BlockSpec((1, GB, BATCH), lambda i: (0, i, 0)),
            pl.BlockSpec((1, GB, BATCH), lambda i: (1, i, 0)),
        ],
        out_specs=pl.BlockSpec((GB, BATCH), lambda i: (i, 0)),
        out_shape=jax.ShapeDtypeStruct((NPAD, BATCH), jnp.float32),
    )(xp, hist, hist)


def _mid_body(a0_ref, a1_ref, xs_ref, h0_ref, h1_ref, w1_ref, b1_ref, w2_ref,
              o_ref):
    i = pl.program_id(0)
    dinv = _dinv(h0_ref[0, :, 0:1], h1_ref[0, :, 0:1])
    u = dinv * (a0_ref[0] + a1_ref[0] + xs_ref[...])
    hmid = _celu(jnp.dot(u, w1_ref[...], preferred_element_type=jnp.float32)
                 + b1_ref[...])
    p = jnp.dot(hmid, w2_ref[...], preferred_element_type=jnp.float32)
    rows = i * GB + lax.broadcasted_iota(jnp.int32, (GB, 1), 0)
    o_ref[...] = jnp.where(rows < N_G, dinv * p, 0.0)


def _mid_call(acc, xs1, hist, W1, b1r, W2):
    grid = (NPAD // GB,)
    mid = W1.shape[1]
    return pl.pallas_call(
        _mid_body,
        grid=grid,
        in_specs=[
            pl.BlockSpec((1, GB, BATCH), lambda i: (0, i, 0)),
            pl.BlockSpec((1, GB, BATCH), lambda i: (1, i, 0)),
            pl.BlockSpec((GB, BATCH), lambda i: (i, 0)),
            pl.BlockSpec((1, GB, BATCH), lambda i: (0, i, 0)),
            pl.BlockSpec((1, GB, BATCH), lambda i: (1, i, 0)),
            pl.BlockSpec((BATCH, mid), lambda i: (0, 0)),
            pl.BlockSpec((1, mid), lambda i: (0, 0)),
            pl.BlockSpec((mid, BATCH), lambda i: (0, 0)),
        ],
        out_specs=pl.BlockSpec((GB, BATCH), lambda i: (i, 0)),
        out_shape=jax.ShapeDtypeStruct((NPAD, BATCH), jnp.float32),
    )(acc, acc, xs1, hist, hist, W1, b1r, W2)


FB = 2000


def _final1_body(a0_ref, a1_ref, xs_ref, h0_ref, h1_ref, b2_ref, w_ref,
                 db_ref, z_ref):
    i = pl.program_id(0)
    dinv = _dinv(h0_ref[0, :, 0:1], h1_ref[0, :, 0:1])
    hg = _celu(dinv * (a0_ref[0] + a1_ref[0] + xs_ref[...]) + b2_ref[...])
    zpart = lax.dot_general(hg, w_ref[...], (((0,), (0,)), ((), ())),
                            preferred_element_type=jnp.float32)

    @pl.when(i == 0)
    def _():
        z_ref[...] = jnp.zeros_like(z_ref)

    z_ref[...] += zpart

    @pl.when(i == (N_G // FB) - 1)
    def _():
        z_ref[...] += db_ref[...]


def _final1_call(acc, xs2, hist, b2r, dr_W, dr_br):
    grid = (N_G // FB,)
    td = dr_W.shape[1]
    return pl.pallas_call(
        _final1_body,
        grid=grid,
        in_specs=[
            pl.BlockSpec((1, FB, BATCH), lambda i: (0, i, 0)),
            pl.BlockSpec((1, FB, BATCH), lambda i: (1, i, 0)),
            pl.BlockSpec((FB, BATCH), lambda i: (i, 0)),
            pl.BlockSpec((1, FB, BATCH), lambda i: (0, i, 0)),
            pl.BlockSpec((1, FB, BATCH), lambda i: (1, i, 0)),
            pl.BlockSpec((1, BATCH), lambda i: (0, 0)),
            pl.BlockSpec((FB, td), lambda i: (i, 0)),
            pl.BlockSpec((1, td), lambda i: (0, 0)),
        ],
        out_specs=pl.BlockSpec((BATCH, td), lambda i: (0, 0)),
        out_shape=jax.ShapeDtypeStruct((BATCH, td), jnp.float32),
    )(acc, acc, xs2, hist, hist, b2r, dr_W, dr_br)


def _final2_body(z_ref, w1_ref, b1_ref, w2_ref, b2_ref, y_ref):
    t = _celu(jnp.dot(z_ref[...], w1_ref[...],
                      preferred_element_type=jnp.float32) + b1_ref[...])
    y_ref[...] = _celu(jnp.dot(t, w2_ref[...],
                               preferred_element_type=jnp.float32) + b2_ref[...])


FB2 = 2048


def _final2_call(z, r1_W, r1_br, r2_W, r2_br):
    grid = (NPAD // FB2,)
    td = z.shape[1]
    rm = r1_W.shape[1]
    r2_Wp = jnp.pad(r2_W, ((0, 0), (0, NPAD - N_G)))
    r2_brp = jnp.pad(r2_br, ((0, 0), (0, NPAD - N_G)))
    yp = pl.pallas_call(
        _final2_body,
        grid=grid,
        in_specs=[
            pl.BlockSpec((BATCH, td), lambda i: (0, 0)),
            pl.BlockSpec((td, rm), lambda i: (0, 0)),
            pl.BlockSpec((1, rm), lambda i: (0, 0)),
            pl.BlockSpec((rm, FB2), lambda i: (0, i)),
            pl.BlockSpec((1, FB2), lambda i: (0, i)),
        ],
        out_specs=pl.BlockSpec((BATCH, FB2), lambda i: (0, i)),
        out_shape=jax.ShapeDtypeStruct((BATCH, NPAD), jnp.float32),
    )(z, r1_W, r1_br, r2_Wp, r2_brp)
    return yp[:, :N_G]


def kernel(x, edge_index, W1, b1, W2, b2, dr_W, dr_b, r1_W, r1_b, r2_W, r2_b):
    xp = jnp.pad(x, ((0, 0), (0, NPAD - N_G)))
    pad_n = EPAD - N_EDGES
    pad_idx = jnp.full((pad_n,), N_G, jnp.int32)
    src_p = jnp.concatenate([edge_index[0], pad_idx]).reshape(NW, NCH, CHUNK)
    dst_p = jnp.concatenate([edge_index[1], pad_idx]).reshape(NW, NCH, CHUNK)

    hist = _hist_sc(dst_p)

    xs1 = _scale_call(xp, hist)
    acc1 = _agg_sc(xs1, src_p, dst_p)
    xs2 = _mid_call(acc1, xs1, hist, W1,
                    b1.reshape(1, -1), W2)
    acc2 = _agg_sc(xs2, src_p, dst_p)

    z = _final1_call(acc2, xs2, hist, b2.reshape(1, -1),
                     dr_W, dr_b.reshape(1, -1))
    y = _final2_call(z, r1_W, r1_b.reshape(1, -1), r2_W, r2_b.reshape(1, -1))
    return (z, y)
